# baseline (speedup 1.0000x reference)
import jax
import jax.numpy as jnp
from jax.experimental import pallas as pl
from jax.experimental.pallas import tpu as pltpu

E = 8
TOPK = 2
D = 1024
DI = 512
EPS = 1e-06

EPG = 2
CHUNK = 256


def _tdot(a, b):
    return jax.lax.dot_general(
        a, b, (((1,), (1,)), ((), ())), preferred_element_type=jnp.float32)


def _moe_body(x_ref, rmsw_ref, rw_ref, wg_ref, wu_ref, wd_ref,
              out_ref, logits_ref,
              xn_ref, cw_ref, wgb_ref, wub_ref, wdb_ref):
    step = pl.program_id(0)
    T = x_ref.shape[0]

    @pl.when(step == 0)
    def _router():
        x = x_ref[...]
        ones = jnp.full((1, D), 1.0, jnp.float32)
        var = jax.lax.dot_general(
            x * x, ones, (((1,), (1,)), ((), ())),
            preferred_element_type=jnp.float32) * (1.0 / D)
        xn = x * jax.lax.rsqrt(var + EPS) * rmsw_ref[...]
        logits = jax.lax.dot_general(
            xn, rw_ref[...], (((1,), (1,)), ((), ())),
            preferred_element_type=jnp.float32)
        logits_ref[...] = logits

        m = jnp.max(logits, axis=1, keepdims=True)
        ex = jnp.exp(logits - m)
        w = ex / jnp.sum(ex, axis=1, keepdims=True)
        iota = jax.lax.broadcasted_iota(jnp.int32, (T, E), 1)
        m1 = jnp.max(w, axis=1, keepdims=True)
        i1 = jnp.min(jnp.where(w == m1, iota, E), axis=1, keepdims=True)
        oh1 = iota == i1
        w2 = jnp.where(oh1, -jnp.inf, w)
        m2 = jnp.max(w2, axis=1, keepdims=True)
        i2 = jnp.min(jnp.where(w2 == m2, iota, E), axis=1, keepdims=True)
        cw_ref[...] = jnp.where(oh1 | (iota == i2), w, 0.0)

        xn_ref[...] = xn.astype(jnp.bfloat16)
        out_ref[...] = jnp.zeros((T, D), jnp.float32)

    wgb_ref[0] = wg_ref[0].astype(jnp.bfloat16)
    wub_ref[0] = wu_ref[0].astype(jnp.bfloat16)

    eio = jax.lax.broadcasted_iota(jnp.int32, (E, EPG), 0)
    oh = jnp.where(eio == step * EPG + jax.lax.broadcasted_iota(
        jnp.int32, (E, EPG), 1), 1.0, 0.0)

    for c in range(T // CHUNK):
        sl = pl.ds(c * CHUNK, CHUNK)
        xb = xn_ref[sl, :]
        wcols = jax.lax.dot_general(
            cw_ref[sl, :], oh, (((1,), (0,)), ((), ())),
            preferred_element_type=jnp.float32)
        acc = None
        for k in range(EPG):
            g = _tdot(xb, wgb_ref[k])
            u = _tdot(xb, wub_ref[k])
            if c == 0 and k == 0:
                wgb_ref[1] = wg_ref[1].astype(jnp.bfloat16)
                wub_ref[1] = wu_ref[1].astype(jnp.bfloat16)
                wdb_ref[0] = wd_ref[0].astype(jnp.bfloat16)
            if c == 0 and k == 1:
                wdb_ref[1] = wd_ref[1].astype(jnp.bfloat16)
            h = ((g * jax.nn.sigmoid(g)) * u
                 * wcols[:, k:k + 1]).astype(jnp.bfloat16)
            d = _tdot(h, wdb_ref[k])
            acc = d if acc is None else acc + d
        out_ref[sl, :] += acc


def kernel(hidden_states, rms_weight, router_w, w_gate, w_up, w_down):
    shape = hidden_states.shape
    T = shape[0] * shape[1]
    x = hidden_states.reshape(T, D).astype(jnp.float32)

    out, logits = pl.pallas_call(
        _moe_body,
        grid=(E // EPG,),
        in_specs=[
            pl.BlockSpec((T, D), lambda s: (0, 0)),
            pl.BlockSpec((1, D), lambda s: (0, 0)),
            pl.BlockSpec((E, D), lambda s: (0, 0)),
            pl.BlockSpec((EPG, DI, D), lambda s: (s, 0, 0)),
            pl.BlockSpec((EPG, DI, D), lambda s: (s, 0, 0)),
            pl.BlockSpec((EPG, D, DI), lambda s: (s, 0, 0)),
        ],
        out_specs=[
            pl.BlockSpec((T, D), lambda s: (0, 0)),
            pl.BlockSpec((T, E), lambda s: (0, 0)),
        ],
        out_shape=[
            jax.ShapeDtypeStruct((T, D), jnp.float32),
            jax.ShapeDtypeStruct((T, E), jnp.float32),
        ],
        scratch_shapes=[
            pltpu.VMEM((T, D), jnp.bfloat16),
            pltpu.VMEM((T, E), jnp.float32),
            pltpu.VMEM((EPG, DI, D), jnp.bfloat16),
            pltpu.VMEM((EPG, DI, D), jnp.bfloat16),
            pltpu.VMEM((EPG, D, DI), jnp.bfloat16),
        ],
    )(x, rms_weight.reshape(1, D), router_w, w_gate, w_up, w_down)
    return out.reshape(shape), logits


# speedup vs baseline: 1.0233x; 1.0233x over previous
import jax
import jax.numpy as jnp
from jax.experimental import pallas as pl
from jax.experimental.pallas import tpu as pltpu

E = 8
TOPK = 2
D = 1024
DI = 512
EPS = 1e-06

EPG = 2
CHUNK = 512


def _tdot(a, b):
    return jax.lax.dot_general(
        a, b, (((1,), (1,)), ((), ())), preferred_element_type=jnp.float32)


def _moe_body(x_ref, rmsw_ref, rw_ref, wg_ref, wu_ref, wd_ref,
              out_ref, logits_ref,
              xn_ref, cw_ref, wgb_ref, wub_ref, wdb_ref):
    step = pl.program_id(0)
    T = x_ref.shape[0]

    @pl.when(step == 0)
    def _router():
        x = x_ref[...]
        ones = jnp.full((1, D), 1.0, jnp.float32)
        var = jax.lax.dot_general(
            x * x, ones, (((1,), (1,)), ((), ())),
            preferred_element_type=jnp.float32) * (1.0 / D)
        xn = x * jax.lax.rsqrt(var + EPS) * rmsw_ref[...]
        logits = jax.lax.dot_general(
            xn, rw_ref[...], (((1,), (1,)), ((), ())),
            preferred_element_type=jnp.float32)
        logits_ref[...] = logits

        m = jnp.max(logits, axis=1, keepdims=True)
        ex = jnp.exp(logits - m)
        w = ex / jnp.sum(ex, axis=1, keepdims=True)
        iota = jax.lax.broadcasted_iota(jnp.int32, (T, E), 1)
        m1 = jnp.max(w, axis=1, keepdims=True)
        i1 = jnp.min(jnp.where(w == m1, iota, E), axis=1, keepdims=True)
        oh1 = iota == i1
        w2 = jnp.where(oh1, -jnp.inf, w)
        m2 = jnp.max(w2, axis=1, keepdims=True)
        i2 = jnp.min(jnp.where(w2 == m2, iota, E), axis=1, keepdims=True)
        cw_ref[...] = jnp.where(oh1 | (iota == i2), w, 0.0)

        xn_ref[...] = xn.astype(jnp.bfloat16)
        out_ref[...] = jnp.zeros((T, D), jnp.float32)

    wgb_ref[0] = wg_ref[0].astype(jnp.bfloat16)
    wub_ref[0] = wu_ref[0].astype(jnp.bfloat16)

    eio = jax.lax.broadcasted_iota(jnp.int32, (E, EPG), 0)
    oh = jnp.where(eio == step * EPG + jax.lax.broadcasted_iota(
        jnp.int32, (E, EPG), 1), 1.0, 0.0)

    for c in range(T // CHUNK):
        sl = pl.ds(c * CHUNK, CHUNK)
        xb = xn_ref[sl, :]
        wcols = jax.lax.dot_general(
            cw_ref[sl, :], oh, (((1,), (0,)), ((), ())),
            preferred_element_type=jnp.float32)
        acc = None
        for k in range(EPG):
            g = _tdot(xb, wgb_ref[k])
            u = _tdot(xb, wub_ref[k])
            if c == 0 and k == 0:
                wgb_ref[1] = wg_ref[1].astype(jnp.bfloat16)
                wub_ref[1] = wu_ref[1].astype(jnp.bfloat16)
                wdb_ref[0] = wd_ref[0].astype(jnp.bfloat16)
            if c == 0 and k == 1:
                wdb_ref[1] = wd_ref[1].astype(jnp.bfloat16)
            h = ((g * jax.nn.sigmoid(g)) * u
                 * wcols[:, k:k + 1]).astype(jnp.bfloat16)
            d = _tdot(h, wdb_ref[k])
            acc = d if acc is None else acc + d
        out_ref[sl, :] += acc


def kernel(hidden_states, rms_weight, router_w, w_gate, w_up, w_down):
    shape = hidden_states.shape
    T = shape[0] * shape[1]
    x = hidden_states.reshape(T, D).astype(jnp.float32)

    out, logits = pl.pallas_call(
        _moe_body,
        grid=(E // EPG,),
        in_specs=[
            pl.BlockSpec((T, D), lambda s: (0, 0)),
            pl.BlockSpec((1, D), lambda s: (0, 0)),
            pl.BlockSpec((E, D), lambda s: (0, 0)),
            pl.BlockSpec((EPG, DI, D), lambda s: (s, 0, 0)),
            pl.BlockSpec((EPG, DI, D), lambda s: (s, 0, 0)),
            pl.BlockSpec((EPG, D, DI), lambda s: (s, 0, 0)),
        ],
        out_specs=[
            pl.BlockSpec((T, D), lambda s: (0, 0)),
            pl.BlockSpec((T, E), lambda s: (0, 0)),
        ],
        out_shape=[
            jax.ShapeDtypeStruct((T, D), jnp.float32),
            jax.ShapeDtypeStruct((T, E), jnp.float32),
        ],
        scratch_shapes=[
            pltpu.VMEM((T, D), jnp.bfloat16),
            pltpu.VMEM((T, E), jnp.float32),
            pltpu.VMEM((EPG, DI, D), jnp.bfloat16),
            pltpu.VMEM((EPG, DI, D), jnp.bfloat16),
            pltpu.VMEM((EPG, D, DI), jnp.bfloat16),
        ],
    )(x, rms_weight.reshape(1, D), router_w, w_gate, w_up, w_down)
    return out.reshape(shape), logits


# speedup vs baseline: 1.0413x; 1.0176x over previous
import jax
import jax.numpy as jnp
from jax.experimental import pallas as pl
from jax.experimental.pallas import tpu as pltpu

E = 8
TOPK = 2
D = 1024
DI = 512
EPS = 1e-06

EPG = 2
CHUNK = 512


def _tdot(a, b):
    return jax.lax.dot_general(
        a, b, (((1,), (1,)), ((), ())), preferred_element_type=jnp.float32)


def _moe_body(x_ref, rmsw_ref, rw_ref, wg_ref, wu_ref, wd_ref,
              out_ref, logits_ref,
              xn_ref, cw_ref, wgb_ref, wub_ref, wdb_ref):
    step = pl.program_id(0)
    T = x_ref.shape[0]

    @pl.when(step == 0)
    def _router():
        x = x_ref[...]
        ones = jnp.full((1, D), 1.0, jnp.float32)
        var = jax.lax.dot_general(
            x * x, ones, (((1,), (1,)), ((), ())),
            preferred_element_type=jnp.float32) * (1.0 / D)
        xn = x * jax.lax.rsqrt(var + EPS) * rmsw_ref[...]
        logits = jax.lax.dot_general(
            xn, rw_ref[...], (((1,), (1,)), ((), ())),
            preferred_element_type=jnp.float32)
        logits_ref[...] = logits

        m = jnp.max(logits, axis=1, keepdims=True)
        ex = jnp.exp(logits - m)
        w = ex / jnp.sum(ex, axis=1, keepdims=True)
        iota = jax.lax.broadcasted_iota(jnp.int32, (T, E), 1)
        m1 = jnp.max(w, axis=1, keepdims=True)
        i1 = jnp.min(jnp.where(w == m1, iota, E), axis=1, keepdims=True)
        oh1 = iota == i1
        w2 = jnp.where(oh1, -jnp.inf, w)
        m2 = jnp.max(w2, axis=1, keepdims=True)
        i2 = jnp.min(jnp.where(w2 == m2, iota, E), axis=1, keepdims=True)
        cw_ref[...] = jnp.where(oh1 | (iota == i2), w, 0.0)

        xn_ref[...] = xn.astype(jnp.bfloat16)
        out_ref[...] = jnp.zeros((T, D), jnp.float32)

    @pl.when(step > 0)
    def _experts():
        wgb_ref[0] = wg_ref[0].astype(jnp.bfloat16)
        wub_ref[0] = wu_ref[0].astype(jnp.bfloat16)

        eio = jax.lax.broadcasted_iota(jnp.int32, (E, EPG), 0)
        oh = jnp.where(eio == (step - 1) * EPG + jax.lax.broadcasted_iota(
            jnp.int32, (E, EPG), 1), 1.0, 0.0)

        for c in range(T // CHUNK):
            sl = pl.ds(c * CHUNK, CHUNK)
            xb = xn_ref[sl, :]
            wcols = jax.lax.dot_general(
                cw_ref[sl, :], oh, (((1,), (0,)), ((), ())),
                preferred_element_type=jnp.float32)
            acc = None
            for k in range(EPG):
                g = _tdot(xb, wgb_ref[k])
                u = _tdot(xb, wub_ref[k])
                if c == 0 and k == 0:
                    wgb_ref[1] = wg_ref[1].astype(jnp.bfloat16)
                    wub_ref[1] = wu_ref[1].astype(jnp.bfloat16)
                    wdb_ref[0] = wd_ref[0].astype(jnp.bfloat16)
                if c == 0 and k == 1:
                    wdb_ref[1] = wd_ref[1].astype(jnp.bfloat16)
                h = ((g * jax.nn.sigmoid(g)) * u
                     * wcols[:, k:k + 1]).astype(jnp.bfloat16)
                d = _tdot(h, wdb_ref[k])
                acc = d if acc is None else acc + d
            out_ref[sl, :] += acc


def kernel(hidden_states, rms_weight, router_w, w_gate, w_up, w_down):
    shape = hidden_states.shape
    T = shape[0] * shape[1]
    x = hidden_states.reshape(T, D).astype(jnp.float32)

    out, logits = pl.pallas_call(
        _moe_body,
        grid=(E // EPG + 1,),
        in_specs=[
            pl.BlockSpec((T, D), lambda s: (0, 0)),
            pl.BlockSpec((1, D), lambda s: (0, 0)),
            pl.BlockSpec((E, D), lambda s: (0, 0)),
            pl.BlockSpec((EPG, DI, D),
                         lambda s: (jnp.maximum(s - 1, 0), 0, 0)),
            pl.BlockSpec((EPG, DI, D),
                         lambda s: (jnp.maximum(s - 1, 0), 0, 0)),
            pl.BlockSpec((EPG, D, DI),
                         lambda s: (jnp.maximum(s - 1, 0), 0, 0)),
        ],
        out_specs=[
            pl.BlockSpec((T, D), lambda s: (0, 0)),
            pl.BlockSpec((T, E), lambda s: (0, 0)),
        ],
        out_shape=[
            jax.ShapeDtypeStruct((T, D), jnp.float32),
            jax.ShapeDtypeStruct((T, E), jnp.float32),
        ],
        scratch_shapes=[
            pltpu.VMEM((T, D), jnp.bfloat16),
            pltpu.VMEM((T, E), jnp.float32),
            pltpu.VMEM((EPG, DI, D), jnp.bfloat16),
            pltpu.VMEM((EPG, DI, D), jnp.bfloat16),
            pltpu.VMEM((EPG, D, DI), jnp.bfloat16),
        ],
    )(x, rms_weight.reshape(1, D), router_w, w_gate, w_up, w_down)
    return out.reshape(shape), logits
